# baseline (speedup 1.0000x reference)
import functools

import jax
import jax.numpy as jnp
from jax import lax
from jax.experimental import pallas as pl
from jax.experimental.pallas import tpu as pltpu
from jax.experimental.pallas import tpu_sc as plsc

NUM_WORDS = 1000000
D = 64
B = 16384
NC, NS, L = 2, 16, 16
NW = NC * NS
BPW = B // NW
GROUPS = BPW // L


def _glove_sc(ci_hbm, xi_hbm, wc_hbm, wx_hbm, bc_hbm, bx_hbm, out_hbm,
              ci_v, xi_v, wc_v, wx_v, bc_v, bx_v, out_v, sem):
    wid = lax.axis_index("s") * NC + lax.axis_index("c")
    base = wid * BPW

    pltpu.sync_copy(ci_hbm.at[pl.ds(base, BPW)], ci_v)
    pltpu.sync_copy(xi_hbm.at[pl.ds(base, BPW)], xi_v)

    c1 = pltpu.async_copy(wc_hbm.at[ci_v], wc_v, sem)
    c2 = pltpu.async_copy(wx_hbm.at[xi_v], wx_v, sem)
    c3 = pltpu.async_copy(bc_hbm.at[ci_v], bc_v, sem)
    c4 = pltpu.async_copy(bx_hbm.at[xi_v], bx_v, sem)
    c1.wait()
    c2.wait()
    c3.wait()
    c4.wait()

    lane = lax.iota(jnp.int32, L)

    def group(g, carry):
        rows = g * L + lane
        acc = bc_v[pl.ds(g * L, L)] + bx_v[pl.ds(g * L, L)]
        for d in range(D):
            col = jnp.full((L,), d, jnp.int32)
            a = plsc.load_gather(wc_v, [rows, col])
            b = plsc.load_gather(wx_v, [rows, col])
            acc = acc + a * b
        out_v[pl.ds(g * L, L)] = acc
        return carry

    lax.fori_loop(0, GROUPS, group, 0)

    pltpu.sync_copy(out_v, out_hbm.at[pl.ds(base, BPW)])


@jax.jit
def _launch(ci, xi, w_center, w_contex, b_center, b_contex):
    mesh = plsc.VectorSubcoreMesh(core_axis_name="c", subcore_axis_name="s")
    run = pl.kernel(
        _glove_sc,
        out_type=jax.ShapeDtypeStruct((B,), jnp.float32),
        mesh=mesh,
        scratch_types=[
            pltpu.VMEM((BPW,), jnp.int32),
            pltpu.VMEM((BPW,), jnp.int32),
            pltpu.VMEM((BPW, D), jnp.float32),
            pltpu.VMEM((BPW, D), jnp.float32),
            pltpu.VMEM((BPW,), jnp.float32),
            pltpu.VMEM((BPW,), jnp.float32),
            pltpu.VMEM((BPW,), jnp.float32),
            pltpu.SemaphoreType.DMA,
        ],
        compiler_params=pltpu.CompilerParams(
            needs_layout_passes=False, use_tc_tiling_on_sc=False
        ),
    )
    return run(ci, xi, w_center, w_contex, b_center, b_contex)


def kernel(indices, w_center, w_contex, b_center, b_contex):
    ci = indices[:, 0].astype(jnp.int32)
    xi = indices[:, 1].astype(jnp.int32)
    return _launch(ci, xi, w_center, w_contex, b_center, b_contex)
